# baseline (speedup 1.0000x reference)
import functools

import jax
import jax.numpy as jnp
from jax import lax
from jax.experimental import pallas as pl
from jax.experimental.pallas import tpu as pltpu
from jax.experimental.pallas import tpu_sc as plsc

B = 2
N = 4096
CIN = 128
CH = 32
RB = 1024
NRB = N // RB
F32 = jnp.float32


def _dot(a, b, dims):
    return lax.dot_general(a, b, (dims, ((), ())), preferred_element_type=F32)


def _prep0_body(x_ref, wp_ref, bp_ref, wpp_ref, f0_ref, f0t_ref):
    x = x_ref[...]
    f0_ref[...] = jnp.maximum(
        _dot(wp_ref[...], x, ((1,), (0,))) + bp_ref[...][:CH][:, None], 0.0)
    f0t_ref[...] = jnp.maximum(
        _dot(x, wpp_ref[...], ((0,), (1,))) + bp_ref[...][None, :], 0.0)


def _prep0(x, w_pre, b_pad, w_pad):
    return pl.pallas_call(
        _prep0_body,
        out_shape=[
            jax.ShapeDtypeStruct((CH, N), F32),
            jax.ShapeDtypeStruct((N, CIN), F32),
        ],
    )(x, w_pre, b_pad, w_pad)


def _knn_body(k, f_ref, idx_ref, d_ref):
    f = f_ref[...]
    r = pl.program_id(0)
    rows = f_ref[:, pl.ds(r * RB, RB)]
    inner = _dot(rows, f, ((0,), (0,)))
    xx = jnp.sum(f * f, axis=0)[None, :]
    xxi = jnp.transpose(jnp.sum(rows * rows, axis=0)[None, :], (1, 0))
    d_ref[...] = (xxi - 2.0 * inner) + xx
    ii = lax.broadcasted_iota(jnp.int32, (RB, N), 1)
    d = d_ref[...]
    outs = []
    for _ in range(k):
        a = jnp.argmin(d, axis=1).astype(jnp.int32)
        outs.append(a)
        d = jnp.where(ii == a[:, None], jnp.float32(jnp.inf), d)
    idx_ref[...] = jnp.stack(outs, axis=0)


def _knn(f, k):
    return pl.pallas_call(
        functools.partial(_knn_body, k),
        grid=(NRB,),
        in_specs=[pl.BlockSpec((CH, N), lambda r: (0, 0))],
        out_specs=pl.BlockSpec((k, RB), lambda r: (0, r)),
        out_shape=jax.ShapeDtypeStruct((k, N), jnp.int32),
        scratch_shapes=[pltpu.VMEM((RB, N), F32)],
    )(f)


def _make_gather(total_rows, d, chunk=128):
    info = plsc.get_sparse_core_info()
    nw = info.num_cores * info.num_subcores
    per_w = total_rows // nw
    n_chunks = per_w // chunk
    mesh = plsc.VectorSubcoreMesh(core_axis_name="c", subcore_axis_name="s")

    @functools.partial(
        pl.kernel,
        out_type=jax.ShapeDtypeStruct((total_rows, d), F32),
        mesh=mesh,
        scratch_types=[
            pltpu.VMEM((chunk,), jnp.int32),
            pltpu.VMEM((chunk, d), F32),
            pltpu.SemaphoreType.DMA,
        ],
    )
    def gk(table_hbm, idx_hbm, out_hbm, idx_v, rows_v, sem):
        wid = lax.axis_index("s") * info.num_cores + lax.axis_index("c")
        base = wid * per_w
        for c in range(n_chunks):
            off = base + c * chunk
            pltpu.sync_copy(idx_hbm.at[pl.ds(off, chunk)], idx_v)
            pltpu.async_copy(table_hbm.at[idx_v], rows_v, sem).wait()
            pltpu.sync_copy(rows_v, out_hbm.at[pl.ds(off, chunk)])

    return gk


def _gather_rows(table, idx_flat):
    return _make_gather(idx_flat.shape[0], table.shape[1])(table, idx_flat)


def _edge0_body(nbr_ref, ctr_ref, wa_ref, ba_ref, wb_ref, bb_ref,
                wp1_ref, bp1_ref, wp1p_ref, f2_ref, f2t_ref, k0):
    ctr = jnp.broadcast_to(
        ctr_ref[...][:, :CH][:, None, :], (RB, k0, CH)).reshape(RB * k0, CH)
    nbr = nbr_ref[...][:, :CH]
    e = jnp.concatenate([nbr - ctr, ctr], axis=1)
    h = jnp.maximum(
        _dot(e, wa_ref[...], ((1,), (1,))) + ba_ref[...][None, :], 0.0)
    e2 = jnp.maximum(
        _dot(h, wb_ref[...], ((1,), (1,))) + bb_ref[...][None, :], 0.0)
    f1 = jnp.max(e2.reshape(RB, k0, 128), axis=1)
    f2_ref[...] = jnp.maximum(
        _dot(wp1_ref[...], f1, ((1,), (1,))) + bp1_ref[...][:CH][:, None], 0.0)
    f2t_ref[...] = jnp.maximum(
        _dot(f1, wp1p_ref[...], ((1,), (1,))) + bp1_ref[...][None, :], 0.0)


def _edge0(nbr0, f0t, w_e0a, b_e0a, w_e0b, b_e0b, w_pre1, b_pad, w_pad, k0):
    return pl.pallas_call(
        functools.partial(_edge0_body, k0=k0),
        grid=(NRB,),
        in_specs=[
            pl.BlockSpec((RB * k0, CIN), lambda r: (r, 0)),
            pl.BlockSpec((RB, CIN), lambda r: (r, 0)),
            pl.BlockSpec(w_e0a.shape, lambda r: (0, 0)),
            pl.BlockSpec(b_e0a.shape, lambda r: (0,)),
            pl.BlockSpec(w_e0b.shape, lambda r: (0, 0)),
            pl.BlockSpec(b_e0b.shape, lambda r: (0,)),
            pl.BlockSpec(w_pre1.shape, lambda r: (0, 0)),
            pl.BlockSpec((CIN,), lambda r: (0,)),
            pl.BlockSpec((CIN, CIN), lambda r: (0, 0)),
        ],
        out_specs=[
            pl.BlockSpec((CH, RB), lambda r: (0, r)),
            pl.BlockSpec((RB, CIN), lambda r: (r, 0)),
        ],
        out_shape=[
            jax.ShapeDtypeStruct((CH, N), F32),
            jax.ShapeDtypeStruct((N, CIN), F32),
        ],
    )(nbr0, f0t, w_e0a, b_e0a, w_e0b, b_e0b, w_pre1, b_pad, w_pad)


def _edge1dec_body(nbr_ref, ctr_ref, we_ref, be_ref, wd0_ref, bd0_ref,
                   wd1_ref, bd1_ref, wd2_ref, bd2_ref, out_ref, k1):
    ctr = jnp.broadcast_to(
        ctr_ref[...][:, :CH][:, None, :], (RB, k1, CH)).reshape(RB * k1, CH)
    e = jnp.concatenate([nbr_ref[...][:, :CH] - ctr, ctr], axis=1)
    h = jnp.maximum(
        _dot(e, we_ref[...], ((1,), (1,))) + be_ref[...][None, :], 0.0)
    f3 = jnp.sum(h.reshape(RB, k1, 128), axis=1)
    g0 = jnp.maximum(
        _dot(f3, wd0_ref[...], ((1,), (1,))) + bd0_ref[...][None, :], 0.0)
    g1 = jnp.maximum(
        _dot(g0, wd1_ref[...], ((1,), (1,))) + bd1_ref[...][None, :], 0.0)
    g2 = jnp.maximum(
        _dot(g1, wd2_ref[...], ((1,), (1,))) + bd2_ref[0, 0], 0.0)
    out_ref[...] = g2[:, :1]


def _edge1dec(nbr1, f2t, w_e1, b_e1, w_d0, b_d0, w_d1, b_d1, w_d2, b_d2, k1):
    return pl.pallas_call(
        functools.partial(_edge1dec_body, k1=k1),
        grid=(NRB,),
        in_specs=[
            pl.BlockSpec((RB * k1, CIN), lambda r: (r, 0)),
            pl.BlockSpec((RB, CIN), lambda r: (r, 0)),
            pl.BlockSpec(w_e1.shape, lambda r: (0, 0)),
            pl.BlockSpec(b_e1.shape, lambda r: (0,)),
            pl.BlockSpec(w_d0.shape, lambda r: (0, 0)),
            pl.BlockSpec(b_d0.shape, lambda r: (0,)),
            pl.BlockSpec(w_d1.shape, lambda r: (0, 0)),
            pl.BlockSpec(b_d1.shape, lambda r: (0,)),
            pl.BlockSpec(w_d2.shape, lambda r: (0, 0)),
            pl.BlockSpec((1, 1), lambda r: (0, 0)),
        ],
        out_specs=pl.BlockSpec((RB, 1), lambda r: (r, 0)),
        out_shape=jax.ShapeDtypeStruct((N, 1), F32),
    )(nbr1, f2t, w_e1, b_e1, w_d0, b_d0, w_d1, b_d1, w_d2, b_d2)


def kernel(feature, W_pre0, b_pre0, W_e0a, b_e0a, W_e0b, b_e0b,
           W_pre1, b_pre1, W_e1, b_e1, W_d0, b_d0, W_d1, b_d1, W_d2, b_d2):
    x = feature[..., 0]
    zc = jnp.zeros((CIN - CH, CIN), F32)
    zb = jnp.zeros((CIN - CH,), F32)
    wp0_pad = jnp.concatenate([W_pre0, zc], axis=0)
    bp0_pad = jnp.concatenate([b_pre0, zb])
    wp1_pad = jnp.concatenate([W_pre1, zc], axis=0)
    bp1_pad = jnp.concatenate([b_pre1, zb])
    w_d2p = jnp.concatenate([W_d2, jnp.zeros((7, 32), F32)], axis=0)
    b_d2r = b_d2.reshape(1, 1)

    outs = []
    for b in range(B):
        f0, f0t = _prep0(x[b], W_pre0, bp0_pad, wp0_pad)
        idx0 = _knn(f0, 12)
        nbr0 = _gather_rows(f0t, jnp.transpose(idx0, (1, 0)).reshape(-1))
        f2, f2t = _edge0(nbr0, f0t, W_e0a, b_e0a, W_e0b, b_e0b,
                         W_pre1, bp1_pad, wp1_pad, 12)
        idx1 = _knn(f2, 8)
        nbr1 = _gather_rows(f2t, jnp.transpose(idx1, (1, 0)).reshape(-1))
        outs.append(_edge1dec(nbr1, f2t, W_e1, b_e1, W_d0, b_d0,
                              W_d1, b_d1, w_d2p, b_d2r, 8))
    return jnp.stack(outs, axis=0)


# speedup vs baseline: 1.1993x; 1.1993x over previous
import functools

import jax
import jax.numpy as jnp
from jax import lax
from jax.experimental import pallas as pl
from jax.experimental.pallas import tpu as pltpu
from jax.experimental.pallas import tpu_sc as plsc

B = 2
N = 4096
CIN = 128
CH = 32
RB = 512
NRB = N // RB
F32 = jnp.float32


def _dot(a, b, dims):
    return lax.dot_general(a, b, (dims, ((), ())), preferred_element_type=F32)


def _prep0_body(x_ref, wp_ref, bp_ref, wpp_ref, f0_ref, f0t_ref):
    x = x_ref[...]
    f0_ref[...] = jnp.maximum(
        _dot(wp_ref[...], x, ((1,), (0,))) + bp_ref[...][:CH][:, None], 0.0)
    f0t_ref[...] = jnp.maximum(
        _dot(x, wpp_ref[...], ((0,), (1,))) + bp_ref[...][None, :], 0.0)


def _prep0(x, w_pre, b_pad, w_pad):
    return pl.pallas_call(
        _prep0_body,
        out_shape=[
            jax.ShapeDtypeStruct((CH, N), F32),
            jax.ShapeDtypeStruct((N, CIN), F32),
        ],
    )(x, w_pre, b_pad, w_pad)


def _knn_body(k, f_ref, idx_ref, d_ref):
    f = f_ref[...]
    r = pl.program_id(0)
    rows = f_ref[:, pl.ds(r * RB, RB)]
    inner = _dot(rows, f, ((0,), (0,)))
    xx = jnp.sum(f * f, axis=0)[None, :]
    xxi = jnp.transpose(jnp.sum(rows * rows, axis=0)[None, :], (1, 0))
    ii = lax.broadcasted_iota(jnp.int32, (RB, N), 1)
    d = (xxi - 2.0 * inner) + xx
    outs = []
    for _ in range(k):
        a = jnp.argmin(d, axis=1).astype(jnp.int32)
        outs.append(a)
        d = jnp.where(ii == a[:, None], jnp.float32(jnp.inf), d)
    idx_ref[...] = jnp.stack(outs, axis=0)


def _knn(f, k):
    return pl.pallas_call(
        functools.partial(_knn_body, k),
        grid=(NRB,),
        in_specs=[pl.BlockSpec((CH, N), lambda r: (0, 0))],
        out_specs=pl.BlockSpec((k, RB), lambda r: (0, r)),
        out_shape=jax.ShapeDtypeStruct((k, N), jnp.int32),
        scratch_shapes=[pltpu.VMEM((RB, N), F32)],
    )(f)


def _make_gather(total_rows, d, chunk=128):
    info = plsc.get_sparse_core_info()
    nw = info.num_cores * info.num_subcores
    per_w = total_rows // nw
    n_chunks = per_w // chunk
    mesh = plsc.VectorSubcoreMesh(core_axis_name="c", subcore_axis_name="s")

    @functools.partial(
        pl.kernel,
        out_type=jax.ShapeDtypeStruct((total_rows, d), F32),
        mesh=mesh,
        scratch_types=[
            pltpu.VMEM((chunk,), jnp.int32),
            pltpu.VMEM((chunk, d), F32),
            pltpu.SemaphoreType.DMA,
        ],
    )
    def gk(table_hbm, idx_hbm, out_hbm, idx_v, rows_v, sem):
        wid = lax.axis_index("s") * info.num_cores + lax.axis_index("c")
        base = wid * per_w
        for c in range(n_chunks):
            off = base + c * chunk
            pltpu.sync_copy(idx_hbm.at[pl.ds(off, chunk)], idx_v)
            pltpu.async_copy(table_hbm.at[idx_v], rows_v, sem).wait()
            pltpu.sync_copy(rows_v, out_hbm.at[pl.ds(off, chunk)])

    return gk


def _gather_rows(table, idx_flat):
    return _make_gather(idx_flat.shape[0], table.shape[1])(table, idx_flat)


def _edge0_body(nbr_ref, ctr_ref, wa_ref, ba_ref, wb_ref, bb_ref,
                wp1_ref, bp1_ref, wp1p_ref, f2_ref, f2t_ref, k0):
    ctr = jnp.broadcast_to(
        ctr_ref[...][:, :CH][:, None, :], (RB, k0, CH)).reshape(RB * k0, CH)
    nbr = nbr_ref[...][:, :CH]
    e = jnp.concatenate([nbr - ctr, ctr], axis=1)
    h = jnp.maximum(
        _dot(e, wa_ref[...], ((1,), (1,))) + ba_ref[...][None, :], 0.0)
    e2 = jnp.maximum(
        _dot(h, wb_ref[...], ((1,), (1,))) + bb_ref[...][None, :], 0.0)
    f1 = jnp.max(e2.reshape(RB, k0, 128), axis=1)
    f2_ref[...] = jnp.maximum(
        _dot(wp1_ref[...], f1, ((1,), (1,))) + bp1_ref[...][:CH][:, None], 0.0)
    f2t_ref[...] = jnp.maximum(
        _dot(f1, wp1p_ref[...], ((1,), (1,))) + bp1_ref[...][None, :], 0.0)


def _edge0(nbr0, f0t, w_e0a, b_e0a, w_e0b, b_e0b, w_pre1, b_pad, w_pad, k0):
    return pl.pallas_call(
        functools.partial(_edge0_body, k0=k0),
        grid=(NRB,),
        in_specs=[
            pl.BlockSpec((RB * k0, CIN), lambda r: (r, 0)),
            pl.BlockSpec((RB, CIN), lambda r: (r, 0)),
            pl.BlockSpec(w_e0a.shape, lambda r: (0, 0)),
            pl.BlockSpec(b_e0a.shape, lambda r: (0,)),
            pl.BlockSpec(w_e0b.shape, lambda r: (0, 0)),
            pl.BlockSpec(b_e0b.shape, lambda r: (0,)),
            pl.BlockSpec(w_pre1.shape, lambda r: (0, 0)),
            pl.BlockSpec((CIN,), lambda r: (0,)),
            pl.BlockSpec((CIN, CIN), lambda r: (0, 0)),
        ],
        out_specs=[
            pl.BlockSpec((CH, RB), lambda r: (0, r)),
            pl.BlockSpec((RB, CIN), lambda r: (r, 0)),
        ],
        out_shape=[
            jax.ShapeDtypeStruct((CH, N), F32),
            jax.ShapeDtypeStruct((N, CIN), F32),
        ],
    )(nbr0, f0t, w_e0a, b_e0a, w_e0b, b_e0b, w_pre1, b_pad, w_pad)


def _edge1dec_body(nbr_ref, ctr_ref, we_ref, be_ref, wd0_ref, bd0_ref,
                   wd1_ref, bd1_ref, wd2_ref, bd2_ref, out_ref, k1):
    ctr = jnp.broadcast_to(
        ctr_ref[...][:, :CH][:, None, :], (RB, k1, CH)).reshape(RB * k1, CH)
    e = jnp.concatenate([nbr_ref[...][:, :CH] - ctr, ctr], axis=1)
    h = jnp.maximum(
        _dot(e, we_ref[...], ((1,), (1,))) + be_ref[...][None, :], 0.0)
    f3 = jnp.sum(h.reshape(RB, k1, 128), axis=1)
    g0 = jnp.maximum(
        _dot(f3, wd0_ref[...], ((1,), (1,))) + bd0_ref[...][None, :], 0.0)
    g1 = jnp.maximum(
        _dot(g0, wd1_ref[...], ((1,), (1,))) + bd1_ref[...][None, :], 0.0)
    g2 = jnp.maximum(
        _dot(g1, wd2_ref[...], ((1,), (1,))) + bd2_ref[0, 0], 0.0)
    out_ref[...] = g2[:, :1]


def _edge1dec(nbr1, f2t, w_e1, b_e1, w_d0, b_d0, w_d1, b_d1, w_d2, b_d2, k1):
    return pl.pallas_call(
        functools.partial(_edge1dec_body, k1=k1),
        grid=(NRB,),
        in_specs=[
            pl.BlockSpec((RB * k1, CIN), lambda r: (r, 0)),
            pl.BlockSpec((RB, CIN), lambda r: (r, 0)),
            pl.BlockSpec(w_e1.shape, lambda r: (0, 0)),
            pl.BlockSpec(b_e1.shape, lambda r: (0,)),
            pl.BlockSpec(w_d0.shape, lambda r: (0, 0)),
            pl.BlockSpec(b_d0.shape, lambda r: (0,)),
            pl.BlockSpec(w_d1.shape, lambda r: (0, 0)),
            pl.BlockSpec(b_d1.shape, lambda r: (0,)),
            pl.BlockSpec(w_d2.shape, lambda r: (0, 0)),
            pl.BlockSpec((1, 1), lambda r: (0, 0)),
        ],
        out_specs=pl.BlockSpec((RB, 1), lambda r: (r, 0)),
        out_shape=jax.ShapeDtypeStruct((N, 1), F32),
    )(nbr1, f2t, w_e1, b_e1, w_d0, b_d0, w_d1, b_d1, w_d2, b_d2)


def kernel(feature, W_pre0, b_pre0, W_e0a, b_e0a, W_e0b, b_e0b,
           W_pre1, b_pre1, W_e1, b_e1, W_d0, b_d0, W_d1, b_d1, W_d2, b_d2):
    x = feature[..., 0]
    zc = jnp.zeros((CIN - CH, CIN), F32)
    zb = jnp.zeros((CIN - CH,), F32)
    wp0_pad = jnp.concatenate([W_pre0, zc], axis=0)
    bp0_pad = jnp.concatenate([b_pre0, zb])
    wp1_pad = jnp.concatenate([W_pre1, zc], axis=0)
    bp1_pad = jnp.concatenate([b_pre1, zb])
    w_d2p = jnp.concatenate([W_d2, jnp.zeros((7, 32), F32)], axis=0)
    b_d2r = b_d2.reshape(1, 1)

    outs = []
    for b in range(B):
        f0, f0t = _prep0(x[b], W_pre0, bp0_pad, wp0_pad)
        idx0 = _knn(f0, 12)
        nbr0 = _gather_rows(f0t, jnp.transpose(idx0, (1, 0)).reshape(-1))
        f2, f2t = _edge0(nbr0, f0t, W_e0a, b_e0a, W_e0b, b_e0b,
                         W_pre1, bp1_pad, wp1_pad, 12)
        idx1 = _knn(f2, 8)
        nbr1 = _gather_rows(f2t, jnp.transpose(idx1, (1, 0)).reshape(-1))
        outs.append(_edge1dec(nbr1, f2t, W_e1, b_e1, W_d0, b_d0,
                              W_d1, b_d1, w_d2p, b_d2r, 8))
    return jnp.stack(outs, axis=0)
